# Initial kernel scaffold; baseline (speedup 1.0000x reference)
import jax
import jax.numpy as jnp
from jax.experimental import pallas as pl


def kernel(skill_ids_1, skill_ids_2, table, W1, b1, W2, b2):
    raise NotImplementedError("write your pallas kernel here")


# speedup vs baseline: 9.5939x; 9.5939x over previous
import functools

import jax
import jax.numpy as jnp
from jax import lax
from jax.experimental import pallas as pl
from jax.experimental.pallas import tpu as pltpu
from jax.experimental.pallas import tpu_sc as plsc

BATCH = 16384
NUM_SKILLS = 100000
SKILL_DIM = 64
HIDDEN_DIM = 128
LIST_LEN = 20

NUM_CORES = 2
NUM_SUBCORES = 16
NW = NUM_CORES * NUM_SUBCORES

CHUNK = 32
IDS_PER_CHUNK = CHUNK * 2 * LIST_LEN
IDX_ROWS = IDS_PER_CHUNK // 128
CHUNKS_PER_W = BATCH // (NW * CHUNK)
TOTAL_CHUNKS = BATCH // CHUNK


def _pooling_sc(table, idx3d):
    mesh = plsc.VectorSubcoreMesh(
        core_axis_name="c", subcore_axis_name="s",
        num_cores=NUM_CORES, num_subcores=NUM_SUBCORES)

    @functools.partial(
        pl.kernel,
        out_type=jax.ShapeDtypeStruct((BATCH, 2 * SKILL_DIM), jnp.float32),
        mesh=mesh,
        scratch_types=[
            pltpu.VMEM((IDX_ROWS, 128), jnp.int32),
            pltpu.VMEM((IDS_PER_CHUNK, SKILL_DIM), jnp.float32),
            pltpu.VMEM((CHUNK, 2 * SKILL_DIM), jnp.float32),
            pltpu.SemaphoreType.DMA,
        ],
        compiler_params=pltpu.CompilerParams(use_tc_tiling_on_sc=False),
    )
    def k(table_hbm, idx_hbm, out_hbm, idx_v, rows_v, out_v, sem):
        wid = lax.axis_index("s") * NUM_CORES + lax.axis_index("c")

        @pl.loop(0, CHUNKS_PER_W)
        def _chunk(c):
            kk = wid * CHUNKS_PER_W + c
            pltpu.sync_copy(idx_hbm.at[kk], idx_v)
            cps = [
                pltpu.async_copy(
                    table_hbm.at[idx_v.at[j]],
                    rows_v.at[pl.ds(j * 128, 128)], sem)
                for j in range(IDX_ROWS)
            ]
            for cp in cps:
                cp.wait()

            @pl.loop(0, CHUNK)
            def _elem(i):
                row0 = i * (2 * LIST_LEN)
                for half in range(2):
                    base = row0 + half * LIST_LEN
                    for d in range(SKILL_DIM // 16):
                        acc = rows_v[base, pl.ds(d * 16, 16)]
                        for r in range(1, LIST_LEN):
                            acc = acc + rows_v[base + r, pl.ds(d * 16, 16)]
                        out_v[i, pl.ds(half * SKILL_DIM + d * 16, 16)] = acc

            pltpu.sync_copy(out_v, out_hbm.at[pl.ds(kk * CHUNK, CHUNK)])

    return k(table, idx3d)


def _mlp_body(x_ref, w1t_ref, b1_ref, w2_ref, b2_ref, o_ref):
    h = jnp.dot(x_ref[...], w1t_ref[...], preferred_element_type=jnp.float32)
    h = jnp.maximum(h + b1_ref[...], 0.0)
    z = jnp.sum(h * w2_ref[...], axis=1, keepdims=True) + b2_ref[...]
    o_ref[...] = 1.0 / (1.0 + jnp.exp(-z))


def _mlp_tc(x, w1t, b1, w2, b2):
    BM = 1024
    grid = (BATCH // BM,)
    return pl.pallas_call(
        _mlp_body,
        grid=grid,
        in_specs=[
            pl.BlockSpec((BM, 2 * SKILL_DIM), lambda i: (i, 0)),
            pl.BlockSpec((2 * SKILL_DIM, HIDDEN_DIM), lambda i: (0, 0)),
            pl.BlockSpec((1, HIDDEN_DIM), lambda i: (0, 0)),
            pl.BlockSpec((1, HIDDEN_DIM), lambda i: (0, 0)),
            pl.BlockSpec((1, 1), lambda i: (0, 0)),
        ],
        out_specs=pl.BlockSpec((BM, 1), lambda i: (i, 0)),
        out_shape=jax.ShapeDtypeStruct((BATCH, 1), jnp.float32),
    )(x, w1t, b1, w2, b2)


def kernel(skill_ids_1, skill_ids_2, table, W1, b1, W2, b2):
    ids = jnp.concatenate(
        [skill_ids_1.astype(jnp.int32), skill_ids_2.astype(jnp.int32)],
        axis=1)
    idx3d = ids.reshape(TOTAL_CHUNKS, IDX_ROWS, 128)
    pooled = _pooling_sc(table, idx3d)
    w1t = W1.T * (1.0 / LIST_LEN)
    return _mlp_tc(pooled, w1t, b1.reshape(1, -1), W2, b2.reshape(1, 1))


# speedup vs baseline: 11.5155x; 1.2003x over previous
import functools

import jax
import jax.numpy as jnp
from jax import lax
from jax.experimental import pallas as pl
from jax.experimental.pallas import tpu as pltpu
from jax.experimental.pallas import tpu_sc as plsc

BATCH = 16384
NUM_SKILLS = 100000
SKILL_DIM = 64
HIDDEN_DIM = 128
LIST_LEN = 20

NUM_CORES = 2
NUM_SUBCORES = 16
NW = NUM_CORES * NUM_SUBCORES

CHUNK = 16
IDS_PER_CHUNK = CHUNK * 2 * LIST_LEN
IDX_ROWS = IDS_PER_CHUNK // 128
CHUNKS_PER_W = BATCH // (NW * CHUNK)
TOTAL_CHUNKS = BATCH // CHUNK


def _pooling_sc(table, idx3d):
    mesh = plsc.VectorSubcoreMesh(
        core_axis_name="c", subcore_axis_name="s",
        num_cores=NUM_CORES, num_subcores=NUM_SUBCORES)

    @functools.partial(
        pl.kernel,
        out_type=jax.ShapeDtypeStruct((BATCH, 2 * SKILL_DIM), jnp.float32),
        mesh=mesh,
        scratch_types=[
            pltpu.VMEM((IDX_ROWS, 128), jnp.int32),
            pltpu.VMEM((IDX_ROWS, 128), jnp.int32),
            pltpu.VMEM((IDS_PER_CHUNK, SKILL_DIM), jnp.float32),
            pltpu.VMEM((IDS_PER_CHUNK, SKILL_DIM), jnp.float32),
            pltpu.VMEM((CHUNK, 2 * SKILL_DIM), jnp.float32),
            pltpu.VMEM((CHUNK, 2 * SKILL_DIM), jnp.float32),
            pltpu.SemaphoreType.DMA,
            pltpu.SemaphoreType.DMA,
        ],
        compiler_params=pltpu.CompilerParams(use_tc_tiling_on_sc=False),
    )
    def k(table_hbm, idx_hbm, out_hbm,
          idx_a, idx_b, rows_a, rows_b, out_a, out_b, sem_a, sem_b):
        wid = lax.axis_index("s") * NUM_CORES + lax.axis_index("c")
        base = wid * CHUNKS_PER_W

        def fire(kk, idx_v, rows_v, sem):
            pltpu.sync_copy(idx_hbm.at[kk], idx_v)
            for j in range(IDX_ROWS):
                pltpu.async_copy(
                    table_hbm.at[idx_v.at[j]],
                    rows_v.at[pl.ds(j * 128, 128)], sem)

        def drain(idx_v, rows_v, sem):
            for j in range(IDX_ROWS):
                pltpu.make_async_copy(
                    table_hbm.at[idx_v.at[j]],
                    rows_v.at[pl.ds(j * 128, 128)], sem).wait()

        def reduce_store(kk, rows_v, out_v):
            @pl.loop(0, CHUNK)
            def _elem(i):
                row0 = i * (2 * LIST_LEN)
                for half in range(2):
                    rbase = row0 + half * LIST_LEN
                    for d in range(SKILL_DIM // 16):
                        acc = rows_v[rbase, pl.ds(d * 16, 16)]
                        for r in range(1, LIST_LEN):
                            acc = acc + rows_v[rbase + r, pl.ds(d * 16, 16)]
                        out_v[i, pl.ds(half * SKILL_DIM + d * 16, 16)] = acc

            pltpu.sync_copy(out_v, out_hbm.at[pl.ds(kk * CHUNK, CHUNK)])

        fire(base, idx_a, rows_a, sem_a)

        @pl.loop(0, CHUNKS_PER_W // 2)
        def _pair(c2):
            c0 = base + 2 * c2
            fire(c0 + 1, idx_b, rows_b, sem_b)
            drain(idx_a, rows_a, sem_a)
            reduce_store(c0, rows_a, out_a)

            @pl.when(c2 < CHUNKS_PER_W // 2 - 1)
            def _():
                fire(c0 + 2, idx_a, rows_a, sem_a)

            drain(idx_b, rows_b, sem_b)
            reduce_store(c0 + 1, rows_b, out_b)

    return k(table, idx3d)


def _mlp_body(x_ref, w1t_ref, b1_ref, w2_ref, b2_ref, o_ref):
    h = jnp.dot(x_ref[...], w1t_ref[...], preferred_element_type=jnp.float32)
    h = jnp.maximum(h + b1_ref[...], 0.0)
    z = jnp.sum(h * w2_ref[...], axis=1, keepdims=True) + b2_ref[...]
    o_ref[...] = 1.0 / (1.0 + jnp.exp(-z))


def _mlp_tc(x, w1t, b1, w2, b2):
    BM = 1024
    grid = (BATCH // BM,)
    return pl.pallas_call(
        _mlp_body,
        grid=grid,
        in_specs=[
            pl.BlockSpec((BM, 2 * SKILL_DIM), lambda i: (i, 0)),
            pl.BlockSpec((2 * SKILL_DIM, HIDDEN_DIM), lambda i: (0, 0)),
            pl.BlockSpec((1, HIDDEN_DIM), lambda i: (0, 0)),
            pl.BlockSpec((1, HIDDEN_DIM), lambda i: (0, 0)),
            pl.BlockSpec((1, 1), lambda i: (0, 0)),
        ],
        out_specs=pl.BlockSpec((BM, 1), lambda i: (i, 0)),
        out_shape=jax.ShapeDtypeStruct((BATCH, 1), jnp.float32),
    )(x, w1t, b1, w2, b2)


def kernel(skill_ids_1, skill_ids_2, table, W1, b1, W2, b2):
    ids = jnp.concatenate(
        [skill_ids_1.astype(jnp.int32), skill_ids_2.astype(jnp.int32)],
        axis=1)
    idx3d = ids.reshape(TOTAL_CHUNKS, IDX_ROWS, 128)
    pooled = _pooling_sc(table, idx3d)
    w1t = W1.T * (1.0 / LIST_LEN)
    return _mlp_tc(pooled, w1t, b1.reshape(1, -1), W2, b2.reshape(1, 1))
